# baseline (speedup 1.0000x reference)
import functools

import jax
import jax.numpy as jnp
from jax import lax
from jax.experimental import pallas as pl
from jax.experimental.pallas import tpu as pltpu
from jax.experimental.pallas import tpu_sc as plsc


_ROWS = 8


def _topk_body(k, s_ref, o_ref):
    blk = s_ref[...]
    n = blk.shape[1]
    blk_t = jnp.transpose(blk, (1, 0))
    ii = lax.broadcasted_iota(jnp.int32, (n, n), 0)
    jj = lax.broadcasted_iota(jnp.int32, (n, n), 1)
    jlt = jj < ii
    rr = lax.broadcasted_iota(jnp.int32, (n, k), 1)
    ivals = lax.broadcasted_iota(jnp.int32, (n, k), 0)
    for r in range(_ROWS):
        row = blk[r:r + 1, :]
        col = blk_t[:, r:r + 1]
        win = (row > col) | ((row == col) & jlt)
        rank = jnp.sum(win.astype(jnp.int32), axis=1, keepdims=True)
        onehot = rank == rr
        o_ref[r:r + 1, :] = jnp.sum(
            jnp.where(onehot, ivals, 0), axis=0, keepdims=True)


def _topk_indices(scores, k):
    b, n = scores.shape
    return pl.pallas_call(
        functools.partial(_topk_body, k),
        grid=(b // _ROWS,),
        in_specs=[pl.BlockSpec((_ROWS, n), lambda i: (i, 0))],
        out_specs=pl.BlockSpec((_ROWS, k), lambda i: (i, 0)),
        out_shape=jax.ShapeDtypeStruct((b, k), jnp.int32),
    )(scores)


_CHUNK = 64


def _sc_gather_add(idx, patches_flat, pos_table, k, d):
    b, _ = idx.shape
    n_pos = pos_table.shape[0]
    n_chunks = k // _CHUNK
    mesh = plsc.VectorSubcoreMesh(core_axis_name="c", subcore_axis_name="s")

    @functools.partial(
        pl.kernel,
        mesh=mesh,
        out_type=jax.ShapeDtypeStruct((b * k, d), jnp.float32),
        scratch_types=[
            pltpu.VMEM((k,), jnp.int32),
            pltpu.VMEM((n_chunks, _CHUNK), jnp.int32),
            pltpu.VMEM((2 * n_chunks, _CHUNK // 2), jnp.int32),
            pltpu.VMEM((2, _CHUNK, d), jnp.float32),
            pltpu.VMEM((_CHUNK // 2, d), jnp.float32),
            pltpu.SemaphoreType.DMA,
            pltpu.SemaphoreType.DMA,
            pltpu.SemaphoreType.DMA,
            pltpu.SemaphoreType.DMA,
            pltpu.SemaphoreType.DMA,
        ],
    )
    def sc_kernel(idx_hbm, patches_hbm, pos_hbm, out_hbm,
                  idxrow_v, fidx_v, pidx_v, pbuf, qbuf,
                  sp0, sp1, sq, so0, so1):
        sems_p = (sp0, sp1)
        sems_o = (so0, so1)
        half = _CHUNK // 2
        sid = lax.axis_index("s")
        wid = sid * 2 + lax.axis_index("c")

        pltpu.sync_copy(idx_hbm.at[wid], idxrow_v)
        base = wid * 1024
        for c in range(k // 16):
            v = idxrow_v[pl.ds(c * 16, 16)]
            g = c // (_CHUNK // 16)
            r = (c % (_CHUNK // 16)) * 16
            fidx_v[g, pl.ds(r, 16)] = v + base
            h = r // half
            pidx_v[2 * g + h, pl.ds(r - h * half, 16)] = v + 1

        def start_p(g):
            cp = pltpu.make_async_copy(
                patches_hbm.at[fidx_v.at[g]], pbuf.at[g % 2], sems_p[g % 2])
            cp.start()
            return cp

        def start_q(g, h):
            cq = pltpu.make_async_copy(
                pos_hbm.at[pidx_v.at[2 * g + h]], qbuf, sq)
            cq.start()
            return cq

        def add_half(s, h):
            def body(r, carry):
                for c in range(d // 16):
                    sl = pl.ds(c * 16, 16)
                    plsc.addupdate(pbuf.at[s, h * half + r, sl], qbuf[r, sl])
                return carry
            lax.fori_loop(0, half, body, 0)

        cp = start_p(0)
        owr = {}
        for g in range(n_chunks):
            s = g % 2
            cq = start_q(g, 0)
            cp.wait()
            cq.wait()
            add_half(s, 0)
            cq = start_q(g, 1)
            if g >= 1:
                owr.pop(g - 1).wait()
            if g + 1 < n_chunks:
                cp = start_p(g + 1)
            cq.wait()
            add_half(s, 1)
            co = pltpu.make_async_copy(
                pbuf.at[s],
                out_hbm.at[pl.ds(wid * k + g * _CHUNK, _CHUNK)],
                sems_o[s])
            co.start()
            owr[g] = co
        owr.pop(n_chunks - 1).wait()

    return sc_kernel(idx, patches_flat, pos_table)


def kernel(magno_patches, vit_positional_embedding, scores):
    b, n, d = magno_patches.shape
    k = n // 4
    idx = _topk_indices(scores, k)
    patches_flat = magno_patches.reshape(b * n, d)
    pos_table = vit_positional_embedding.reshape(n + 1, d)
    out = _sc_gather_add(idx, patches_flat, pos_table, k, d)
    return out.reshape(b, k, d)


# speedup vs baseline: 1.0506x; 1.0506x over previous
import functools

import jax
import jax.numpy as jnp
from jax import lax
from jax.experimental import pallas as pl
from jax.experimental.pallas import tpu as pltpu
from jax.experimental.pallas import tpu_sc as plsc


_ROWS = 8


def _topk_body(k, s_ref, o_ref):
    blk = s_ref[...]
    n = blk.shape[1]
    blk_t = jnp.transpose(blk, (1, 0))
    ii = lax.broadcasted_iota(jnp.int32, (n, n), 0)
    jj = lax.broadcasted_iota(jnp.int32, (n, n), 1)
    jlt = jj < ii
    rr = lax.broadcasted_iota(jnp.int32, (n, k), 1)
    ivals = lax.broadcasted_iota(jnp.int32, (n, k), 0)
    for r in range(_ROWS):
        row = blk[r:r + 1, :]
        col = blk_t[:, r:r + 1]
        win = (row > col) | ((row == col) & jlt)
        rank = jnp.sum(win.astype(jnp.int32), axis=1, keepdims=True)
        onehot = rank == rr
        o_ref[r:r + 1, :] = jnp.sum(
            jnp.where(onehot, ivals, 0), axis=0, keepdims=True)


def _topk_indices(scores, k):
    b, n = scores.shape
    return pl.pallas_call(
        functools.partial(_topk_body, k),
        grid=(b // _ROWS,),
        in_specs=[pl.BlockSpec((_ROWS, n), lambda i: (i, 0))],
        out_specs=pl.BlockSpec((_ROWS, k), lambda i: (i, 0)),
        out_shape=jax.ShapeDtypeStruct((b, k), jnp.int32),
    )(scores)


_CHUNK = 64


def _sc_gather_add(idx, patches_flat, pos_table, k, d):
    b, _ = idx.shape
    n_pos = pos_table.shape[0]
    n_chunks = k // _CHUNK
    mesh = plsc.VectorSubcoreMesh(core_axis_name="c", subcore_axis_name="s")

    @functools.partial(
        pl.kernel,
        mesh=mesh,
        out_type=jax.ShapeDtypeStruct((b * k, d), jnp.float32),
        scratch_types=[
            pltpu.VMEM((k,), jnp.int32),
            pltpu.VMEM((n_chunks, _CHUNK), jnp.int32),
            pltpu.VMEM((n_chunks, _CHUNK), jnp.int32),
            pltpu.VMEM((_CHUNK, d), jnp.float32),
            pltpu.VMEM((_CHUNK, d), jnp.float32),
            pltpu.SemaphoreType.DMA,
            pltpu.SemaphoreType.DMA,
            pltpu.SemaphoreType.DMA,
            pltpu.SemaphoreType.DMA,
        ],
    )
    def sc_kernel(idx_hbm, patches_hbm, pos_hbm, out_hbm,
                  idxrow_v, fidx_v, pidx_v, pbuf, qbuf, sp, sq, sw1, sw2):
        half = _CHUNK // 2
        sid = lax.axis_index("s")
        wid = sid * 2 + lax.axis_index("c")

        pltpu.sync_copy(idx_hbm.at[wid], idxrow_v)
        base = wid * 1024
        for c in range(k // 16):
            v = idxrow_v[pl.ds(c * 16, 16)]
            g = c // (_CHUNK // 16)
            r = (c % (_CHUNK // 16)) * 16
            fidx_v[g, pl.ds(r, 16)] = v + base
            pidx_v[g, pl.ds(r, 16)] = v + 1

        def start_p(g):
            cp = pltpu.make_async_copy(patches_hbm.at[fidx_v.at[g]], pbuf, sp)
            cp.start()
            return cp

        def start_q(g):
            cq = pltpu.make_async_copy(pos_hbm.at[pidx_v.at[g]], qbuf, sq)
            cq.start()
            return cq

        def add_half(h):
            def body(r, carry):
                for c in range(d // 16):
                    sl = pl.ds(c * 16, 16)
                    plsc.addupdate(qbuf.at[h * half + r, sl],
                                   pbuf[h * half + r, sl])
                return carry
            lax.fori_loop(0, half, body, 0)

        def start_w(g, h, sem):
            co = pltpu.make_async_copy(
                qbuf.at[pl.ds(h * half, half)],
                out_hbm.at[pl.ds(wid * k + g * _CHUNK + h * half, half)],
                sem)
            co.start()
            return co

        cp = start_p(0)
        cq = start_q(0)
        w1 = w2 = None
        for g in range(n_chunks):
            cp.wait()
            cq.wait()
            add_half(0)
            w1n = start_w(g, 0, sw1)
            add_half(1)
            if g + 1 < n_chunks:
                cp = start_p(g + 1)
            w2n = start_w(g, 1, sw2)
            if g + 1 < n_chunks:
                w1n.wait()
                w2n.wait()
                cq = start_q(g + 1)
            w1, w2 = w1n, w2n
        w1.wait()
        w2.wait()

    return sc_kernel(idx, patches_flat, pos_table)


def kernel(magno_patches, vit_positional_embedding, scores):
    b, n, d = magno_patches.shape
    k = n // 4
    idx = _topk_indices(scores, k)
    patches_flat = magno_patches.reshape(b * n, d)
    pos_table = vit_positional_embedding.reshape(n + 1, d)
    out = _sc_gather_add(idx, patches_flat, pos_table, k, d)
    return out.reshape(b, k, d)
